# baseline (speedup 1.0000x reference)
import jax
import jax.numpy as jnp
from jax.experimental import pallas as pl
from jax.experimental.pallas import tpu as pltpu

_EPS = 1e-4


def _make_fused_kernel(n_total, inv_b):
    inv_nm1 = 1.0 / (n_total - 1)

    def _fused_kernel(xr_ref, xi_ref, gr_ref, gi_ref, br_ref, bi_ref,
                      out_ref, acc_ref, coef_ref):
        p = pl.program_id(0)
        j = pl.program_id(1)

        @pl.when(p == 0)
        def _stats_phase():
            xr = xr_ref[...]
            xi = xi_ref[...]
            tr = jnp.sum(xr, axis=0)
            ti = jnp.sum(xi, axis=0)
            srr = jnp.sum(xr * xr, axis=(0, 1))
            sii = jnp.sum(xi * xi, axis=(0, 1))
            sri = jnp.sum(xr * xi, axis=(0, 1))
            urr = jnp.sum(tr * tr, axis=0)
            uii = jnp.sum(ti * ti, axis=0)
            uri = jnp.sum(tr * ti, axis=0)
            z = jnp.zeros_like(srr)
            upd = jnp.stack([srr, sii, sri, urr, uii, uri, z, z], axis=0)

            @pl.when(j == 0)
            def _():
                acc_ref[...] = upd

            @pl.when(j != 0)
            def _():
                acc_ref[...] += upd

        @pl.when((p == 1) & (j == 0))
        def _coef_phase():
            stats = acc_ref[...]
            srr, sii, sri = stats[0], stats[1], stats[2]
            urr, uii, uri = stats[3], stats[4], stats[5]
            a = (srr - urr * inv_b) * inv_nm1 + _EPS
            c = (sii - uii * inv_b) * inv_nm1 + _EPS
            b = (sri - uri * inv_b) * inv_nm1
            det = a * c - b * b
            s = jnp.sqrt(det)
            k = jax.lax.rsqrt(det * (a + c + 2.0 * s))
            w_rr = (c + s) * k
            w_ii = (a + s) * k
            w_ri = -b * k
            gr = gr_ref[0]
            gi = gi_ref[0]
            coef_ref[0, :] = gr * w_rr - gi * w_ri
            coef_ref[1, :] = gr * w_ri - gi * w_ii
            coef_ref[2, :] = gr * w_ri + gi * w_rr
            coef_ref[3, :] = gr * w_ii + gi * w_ri

        @pl.when(p == 1)
        def _apply_phase():
            crr = coef_ref[0, :]
            cri = coef_ref[1, :]
            cir = coef_ref[2, :]
            cii = coef_ref[3, :]
            xr = xr_ref[...]
            xi = xi_ref[...]
            f = xr.shape[-1]
            mr = jnp.sum(xr, axis=2, keepdims=True) * (1.0 / f)
            mi = jnp.sum(xi, axis=2, keepdims=True) * (1.0 / f)
            xrc = xr - mr
            xic = xi - mi
            out_ref[:, :, 0, :] = crr * xrc + cri * xic + br_ref[0]
            out_ref[:, :, 1, :] = cir * xrc + cii * xic + bi_ref[0]

    return _fused_kernel


def kernel(x_real, x_imag, gamma_r, gamma_i, beta_r, beta_i):
    B, C, F = x_real.shape
    CC = 8
    nc = C // CC

    x_spec = pl.BlockSpec((B, CC, F), lambda p, j: (0, j, 0))
    vec_spec = pl.BlockSpec((1, F), lambda p, j: (0, 0))
    out = pl.pallas_call(
        _make_fused_kernel(B * C, 1.0 / B),
        grid=(2, nc),
        in_specs=[x_spec, x_spec, vec_spec, vec_spec, vec_spec, vec_spec],
        out_specs=pl.BlockSpec(
            (B, CC, 2, F),
            lambda p, j: (0, jnp.where(p == 1, j, 0), 0, 0)),
        out_shape=jax.ShapeDtypeStruct((B, C, 2, F), jnp.float32),
        scratch_shapes=[
            pltpu.VMEM((8, F), jnp.float32),
            pltpu.VMEM((4, F), jnp.float32),
        ],
        compiler_params=pltpu.CompilerParams(
            dimension_semantics=("arbitrary", "arbitrary"),
            vmem_limit_bytes=56 * 1024 * 1024,
        ),
        name="cln_fused",
    )(x_real, x_imag,
      gamma_r.reshape(1, F), gamma_i.reshape(1, F),
      beta_r.reshape(1, F), beta_i.reshape(1, F))

    return out.swapaxes(2, 3)
